# Initial kernel scaffold; baseline (speedup 1.0000x reference)
import jax
import jax.numpy as jnp
from jax.experimental import pallas as pl


def kernel(x, h1, h2, z, edge_index, W0, W1, W2, Wz, Wl, bl, Wm1, bm1, Wm2, bm2):
    raise NotImplementedError("write your pallas kernel here")


# speedup vs baseline: 2.0201x; 2.0201x over previous
import functools

import jax
import jax.numpy as jnp
from jax import lax
from jax.experimental import pallas as pl
from jax.experimental.pallas import tpu as pltpu
from jax.experimental.pallas import tpu_sc as plsc

N = 10000
E = 160000
NSC = 2
NSUB = 16
NW = NSC * NSUB
K = 128
EPW = 5120
NITER = EPW // K
EPAD = NW * EPW
NACC = 10240
ZPT = NACC // NSUB
R = 400
GRID = N // R


def _leaky(v):
    return jnp.where(v > 0, v, 0.2 * v)


def _softmax(t):
    mx = jnp.max(t, axis=1, keepdims=True)
    e = jnp.exp(t - mx)
    return e / jnp.sum(e, axis=1, keepdims=True)


def _l2norm(v):
    n = jnp.sqrt(jnp.sum(v * v, axis=1, keepdims=True))
    return v / jnp.maximum(n, 1e-12)


@functools.lru_cache(maxsize=None)
def _make_segsum(d, nchunks):
    mesh = plsc.VectorSubcoreMesh(
        core_axis_name="c", subcore_axis_name="s",
        num_cores=NSC, num_subcores=NSUB,
    )
    out_t = jax.ShapeDtypeStruct((nchunks, NSC, NACC, d), jnp.float32)

    @functools.partial(
        pl.kernel,
        out_type=out_t,
        mesh=mesh,
        compiler_params=pltpu.CompilerParams(use_tc_tiling_on_sc=(d == 128)),
        scratch_types=[
            pltpu.VMEM((NITER, K), jnp.int32),
            pltpu.VMEM((NITER, K), jnp.int32),
            pltpu.VMEM((K, d), jnp.float32),
            pltpu.VMEM_SHARED((NACC, d), jnp.float32),
        ],
    )
    def seg(table, srcidx, dst2d, zrs, out, sidx_v, didx_v, rows_v, acc):
        cid = lax.axis_index("c")
        sid = lax.axis_index("s")
        w = cid * NSUB + sid
        pltpu.sync_copy(dst2d.at[pl.ds(w * NITER, NITER)], didx_v)
        for c in range(nchunks):
            pltpu.sync_copy(zrs, acc.at[pl.ds(sid * ZPT, ZPT)])
            pltpu.sync_copy(srcidx.at[c, pl.ds(w * NITER, NITER)], sidx_v)
            plsc.subcore_barrier()

            @pl.loop(0, NITER)
            def _(it):
                pltpu.sync_copy(table.at[sidx_v.at[it]], rows_v)
                pltpu.sync_copy(rows_v, acc.at[didx_v.at[it]], add=True)

            plsc.subcore_barrier()
            pltpu.sync_copy(
                acc.at[pl.ds(sid * ZPT, ZPT)],
                out.at[c, cid, pl.ds(sid * ZPT, ZPT)],
            )
            if c + 1 < nchunks:
                plsc.subcore_barrier()

    return seg


def _dot(a, b):
    return jnp.dot(a, b, preferred_element_type=jnp.float32)


def _agg(p):
    nchunks = p.shape[0]
    return jnp.concatenate([p[c, 0] + p[c, 1] for c in range(nchunks)], axis=1)


def _tc0_body(x, w0, s1_o):
    s1_o[...] = _dot(x[...], w0[...])


def _tc1_body(p1, h1, wm1, bm1, w1, z1_o, s2_o):
    z1 = _leaky(_agg(p1[...]))
    h1b = h1[...]
    t = _leaky(_dot(jnp.concatenate([z1, h1b], axis=1), wm1[...]) + bm1[...])
    m = _l2norm(_softmax(t))
    f2 = m[:, 0:1] * z1 + m[:, 1:2] * h1b
    z1_o[...] = z1
    s2_o[...] = _dot(f2, w1[...])


def _tc2_body(p2, h2, wm2, bm2, w2, z2_o, s3_o):
    z2 = _leaky(_agg(p2[...]))
    h2b = h2[...]
    t = _leaky(_dot(jnp.concatenate([z2, h2b], axis=1), wm2[...]) + bm2[...])
    m = _l2norm(_softmax(t))
    f3 = m[:, 0:1] * z2 + m[:, 1:2] * h2b
    z2_o[...] = z2
    s3_o[...] = _dot(f3, w2[...])


def _tc3_body(p3, z1, z2, zz, wl, bl, wzp, s4_o):
    z3 = _leaky(_agg(p3[...]))
    z1b, z2b, zb = z1[...], z2[...], zz[...]
    t = _leaky(_dot(jnp.concatenate([z1b, z2b, z3, zb], axis=1), wl[...])
               + bl[...])
    u = _l2norm(_softmax(t))
    net_in = jnp.concatenate(
        [u[:, 0:1] * z1b, u[:, 1:2] * z2b, u[:, 2:3] * z3, u[:, 3:4] * zb],
        axis=1)
    s4_o[...] = _dot(net_in, wzp[...])


def _tc4_body(p4, no_o, pr_o):
    t = (p4[0, 0] + p4[0, 1])[:, :10]
    no_o[...] = t
    pr_o[...] = _softmax(t)


def _rb(d):
    return pl.BlockSpec((R, d), lambda i: (i, 0))


def _wb(shape):
    return pl.BlockSpec(shape, lambda i: tuple(0 for _ in shape))


def _pb(nchunks, d):
    return pl.BlockSpec((nchunks, NSC, R, d), lambda i: (0, 0, i, 0))


_tc0 = pl.pallas_call(
    _tc0_body,
    grid=(GRID,),
    in_specs=[_rb(256), _wb((256, 512))],
    out_specs=[_rb(512)],
    out_shape=[jax.ShapeDtypeStruct((N, 512), jnp.float32)],
)

_tc1 = pl.pallas_call(
    _tc1_body,
    grid=(GRID,),
    in_specs=[
        _pb(4, 128), _rb(512), _wb((1024, 2)), _wb((1, 2)), _wb((512, 256)),
    ],
    out_specs=[_rb(512), _rb(256)],
    out_shape=[
        jax.ShapeDtypeStruct((N, 512), jnp.float32),
        jax.ShapeDtypeStruct((N, 256), jnp.float32),
    ],
)

_tc2 = pl.pallas_call(
    _tc2_body,
    grid=(GRID,),
    in_specs=[
        _pb(2, 128), _rb(256), _wb((512, 2)), _wb((1, 2)), _wb((256, 64)),
    ],
    out_specs=[_rb(256), _rb(64)],
    out_shape=[
        jax.ShapeDtypeStruct((N, 256), jnp.float32),
        jax.ShapeDtypeStruct((N, 64), jnp.float32),
    ],
)

_tc3 = pl.pallas_call(
    _tc3_body,
    grid=(GRID,),
    in_specs=[
        _pb(1, 64), _rb(512), _rb(256), _rb(64),
        _wb((896, 4)), _wb((1, 4)), _wb((896, 16)),
    ],
    out_specs=[_rb(16)],
    out_shape=[jax.ShapeDtypeStruct((N, 16), jnp.float32)],
)

_tc4 = pl.pallas_call(
    _tc4_body,
    grid=(GRID,),
    in_specs=[_pb(1, 16)],
    out_specs=[_rb(10), _rb(10)],
    out_shape=[
        jax.ShapeDtypeStruct((N, 10), jnp.float32),
        jax.ShapeDtypeStruct((N, 10), jnp.float32),
    ],
)


def kernel(x, h1, h2, z, edge_index, W0, W1, W2, Wz, Wl, bl, Wm1, bm1, Wm2, bm2):
    dst = edge_index[0]
    src = edge_index[1]
    pad = EPAD - E
    dstp = jnp.concatenate([dst, jnp.full((pad,), N, jnp.int32)])
    dst2d = dstp.reshape(NW * NITER, K)
    srcp = jnp.concatenate([src, jnp.zeros((pad,), jnp.int32)])

    def chunked_src(nchunks):
        return (srcp[None, :] * nchunks
                + jnp.arange(nchunks, dtype=jnp.int32)[:, None]).reshape(
                    nchunks, NW * NITER, K)

    src4 = chunked_src(4)
    src2 = chunked_src(2)
    src1 = srcp.reshape(1, NW * NITER, K)
    z128 = jnp.zeros((ZPT, 128), jnp.float32)
    z64 = jnp.zeros((ZPT, 64), jnp.float32)
    z16 = jnp.zeros((ZPT, 16), jnp.float32)

    wzp = jnp.pad(Wz, ((0, 0), (0, 6)))

    s1 = _tc0(x, W0)[0]
    p1 = _make_segsum(128, 4)(s1.reshape(4 * N, 128), src4, dst2d, z128)
    z1, s2 = _tc1(p1, h1, Wm1, bm1.reshape(1, 2), W1)
    p2 = _make_segsum(128, 2)(s2.reshape(2 * N, 128), src2, dst2d, z128)
    z2, s3 = _tc2(p2, h2, Wm2, bm2.reshape(1, 2), W2)
    p3 = _make_segsum(64, 1)(s3, src1, dst2d, z64)
    s4 = _tc3(p3, z1, z2, z, Wl, bl.reshape(1, 4), wzp)[0]
    p4 = _make_segsum(16, 1)(s4, src1, dst2d, z16)
    net_output, predict = _tc4(p4)
    return (net_output, predict)
